# baseline (speedup 1.0000x reference)
import functools

import jax
import jax.numpy as jnp
from jax import lax
from jax.experimental import pallas as pl
from jax.experimental.pallas import tpu as pltpu
from jax.experimental.pallas import tpu_sc as plsc

H_IMG = 1024
W_IMG = 2048
N_PTS = 1_000_000
NW = 32
WPER = 31_248
TAIL = N_PTS - NW * WPER
CH = 4_464
NCH = WPER // CH
L = 16
UNROLL = 6


def _tec_body(points_hbm, depth_hbm, coef_hbm, out_hbm,
              u_v0, u_v1, v_v0, v_v1, idx_v0, idx_v1, d_v0, d_v1,
              q0_v0, q0_v1, q1_v0, q1_v1, coef_v,
              sem_in0, sem_in1, sem_g0, sem_g1, sem_out0, sem_out1):
    u_v = (u_v0, u_v1)
    v_v = (v_v0, v_v1)
    idx_v = (idx_v0, idx_v1)
    d_v = (d_v0, d_v1)
    q0_v = (q0_v0, q0_v1)
    q1_v = (q1_v0, q1_v1)
    sem_in = (sem_in0, sem_in1)
    sem_g = (sem_g0, sem_g1)
    sem_out = (sem_out0, sem_out1)

    cid = lax.axis_index("c")
    sid = lax.axis_index("s")
    wid = sid * 2 + cid
    w_base = wid * WPER

    pltpu.sync_copy(coef_hbm, coef_v)
    m00 = coef_v[0]
    m01 = coef_v[1]
    m02 = coef_v[2]
    m03 = coef_v[3]
    m10 = coef_v[4]
    m11 = coef_v[5]
    m12 = coef_v[6]
    m13 = coef_v[7]
    m20 = coef_v[8]
    m21 = coef_v[9]
    m22 = coef_v[10]
    m23 = coef_v[11]

    def copyin_start(k):
        b = k & 1
        base = w_base + k * CH
        d1 = pltpu.async_copy(points_hbm.at[pl.ds(base, CH)], u_v[b], sem_in[b])
        d2 = pltpu.async_copy(points_hbm.at[pl.ds(N_PTS + base, CH)],
                              v_v[b], sem_in[b])
        return (d1, d2)

    def idx_stage(k):
        b = k & 1

        @plsc.parallel_loop(0, CH, step=L, unroll=UNROLL)
        def body(o):
            u = u_v[b][pl.ds(o, L)]
            v = v_v[b][pl.ds(o, L)]
            idx_v[b][pl.ds(o, L)] = (v << 11) + u

    def gather_start(k):
        b = k & 1
        return pltpu.async_copy(depth_hbm.at[idx_v[b]], d_v[b], sem_g[b])

    def compute(k):
        b = k & 1
        ones = coef_v[12]

        @plsc.parallel_loop(0, CH, step=L, unroll=UNROLL)
        def body(o):
            pu = u_v[b][pl.ds(o, L)].astype(jnp.float32)
            pv = v_v[b][pl.ds(o, L)].astype(jnp.float32)
            d = d_v[b][pl.ds(o, L)]
            pd0 = pu * d
            pd1 = pv * d
            x = m00 * pd0 + m01 * pd1 + m02 * d + m03
            y = m10 * pd0 + m11 * pd1 + m12 * d + m13
            z = m20 * pd0 + m21 * pd1 + m22 * d + m23
            q0_v[b][pl.ds(o, L)] = x / z
            q1_v[b][pl.ds(o, L)] = y / z

    def copyout_start(k):
        b = k & 1
        base = w_base + k * CH
        d1 = pltpu.async_copy(q0_v[b], out_hbm.at[pl.ds(base, CH)], sem_out[b])
        d2 = pltpu.async_copy(q1_v[b], out_hbm.at[pl.ds(N_PTS + base, CH)],
                              sem_out[b])
        return (d1, d2)

    ins = {}
    gs = {}
    outs = {}
    ins[0] = copyin_start(0)
    for dsc in ins[0]:
        dsc.wait()
    idx_stage(0)
    gs[0] = gather_start(0)
    ins[1] = copyin_start(1)
    for k in range(NCH):
        gs[k].wait()
        if k + 1 < NCH:
            for dsc in ins[k + 1]:
                dsc.wait()
            idx_stage(k + 1)
            gs[k + 1] = gather_start(k + 1)
        if k - 2 >= 0:
            for dsc in outs[k - 2]:
                dsc.wait()
        compute(k)
        outs[k] = copyout_start(k)
        if k + 2 < NCH:
            ins[k + 2] = copyin_start(k + 2)
    for k in (NCH - 2, NCH - 1):
        for dsc in outs[k]:
            dsc.wait()

    @pl.when(wid == NW - 1)
    def _():
        base = NW * WPER
        n = TAIL
        pltpu.sync_copy(points_hbm.at[pl.ds(base, n)], u_v[0].at[pl.ds(0, n)])
        pltpu.sync_copy(points_hbm.at[pl.ds(N_PTS + base, n)],
                        v_v[0].at[pl.ds(0, n)])

        def tbody(i, c):
            o = i * L
            u = u_v[0][pl.ds(o, L)]
            v = v_v[0][pl.ds(o, L)]
            idx_v[0][pl.ds(o, L)] = (v << 11) + u
            return c

        lax.fori_loop(0, n // L, tbody, 0)
        pltpu.async_copy(depth_hbm.at[idx_v[0].at[pl.ds(0, n)]],
                         d_v[0].at[pl.ds(0, n)], sem_g[0]).wait()

        def tbody2(i, c):
            o = i * L
            pu = u_v[0][pl.ds(o, L)].astype(jnp.float32)
            pv = v_v[0][pl.ds(o, L)].astype(jnp.float32)
            d = d_v[0][pl.ds(o, L)]
            pd0 = pu * d
            pd1 = pv * d
            x = m00 * pd0 + m01 * pd1 + m02 * d + m03
            y = m10 * pd0 + m11 * pd1 + m12 * d + m13
            z = m20 * pd0 + m21 * pd1 + m22 * d + m23
            q0_v[0][pl.ds(o, L)] = x / z
            q1_v[0][pl.ds(o, L)] = y / z
            return c

        lax.fori_loop(0, n // L, tbody2, 0)
        pltpu.sync_copy(q0_v[0].at[pl.ds(0, n)], out_hbm.at[pl.ds(base, n)])
        pltpu.sync_copy(q1_v[0].at[pl.ds(0, n)],
                        out_hbm.at[pl.ds(N_PTS + base, n)])


@jax.jit
def _reproject(points, depth_flat, coef):
    f = functools.partial(
        pl.kernel,
        mesh=plsc.VectorSubcoreMesh(core_axis_name="c", subcore_axis_name="s"),
        out_type=jax.ShapeDtypeStruct((2 * N_PTS,), jnp.float32),
        scratch_types=[
            pltpu.VMEM((CH,), jnp.int32),
            pltpu.VMEM((CH,), jnp.int32),
            pltpu.VMEM((CH,), jnp.int32),
            pltpu.VMEM((CH,), jnp.int32),
            pltpu.VMEM((CH,), jnp.int32),
            pltpu.VMEM((CH,), jnp.int32),
            pltpu.VMEM((CH,), jnp.float32),
            pltpu.VMEM((CH,), jnp.float32),
            pltpu.VMEM((CH,), jnp.float32),
            pltpu.VMEM((CH,), jnp.float32),
            pltpu.VMEM((CH,), jnp.float32),
            pltpu.VMEM((CH,), jnp.float32),
            pltpu.VMEM((13, 16), jnp.float32),
            pltpu.SemaphoreType.DMA,
            pltpu.SemaphoreType.DMA,
            pltpu.SemaphoreType.DMA,
            pltpu.SemaphoreType.DMA,
            pltpu.SemaphoreType.DMA,
            pltpu.SemaphoreType.DMA,
        ],
    )(_tec_body)
    return f(points, depth_flat, coef)


def kernel(points, depth_img, odometry):
    M = jnp.linalg.inv(odometry)
    c12 = jnp.concatenate([M[:3, :].reshape(12).astype(jnp.float32),
                           jnp.ones((1,), jnp.float32)])
    coef = jnp.broadcast_to(c12.reshape(13, 1), (13, 16))
    depth_flat = depth_img.reshape(-1)
    out_flat = _reproject(points.reshape(-1), depth_flat, coef)
    return out_flat.reshape(2, N_PTS)


# speedup vs baseline: 1.1164x; 1.1164x over previous
import functools

import jax
import jax.numpy as jnp
from jax import lax
from jax.experimental import pallas as pl
from jax.experimental.pallas import tpu as pltpu
from jax.experimental.pallas import tpu_sc as plsc

H_IMG = 1024
W_IMG = 2048
N_PTS = 1_000_000
NW = 32
WPER = 31_232
REM_BASE = NW * WPER
REM_FULL = 512
REM_TAIL = 64
CH = 7_808
NCH = WPER // CH
L = 16
UNROLL = 6


def _tec_body(points_hbm, depth_hbm, coef_hbm, tail_uv_hbm, out_hbm, tailq_hbm,
              uv_v0, uv_v1, idx_v0, idx_v1, d_v0, d_v1, q_v0, q_v1, coef_v,
              t_q, sem_in0, sem_in1, sem_g0, sem_g1, sem_out0, sem_out1):
    uv_v = (uv_v0, uv_v1)
    idx_v = (idx_v0, idx_v1)
    d_v = (d_v0, d_v1)
    q_v = (q_v0, q_v1)
    sem_in = (sem_in0, sem_in1)
    sem_g = (sem_g0, sem_g1)
    sem_out = (sem_out0, sem_out1)

    cid = lax.axis_index("c")
    sid = lax.axis_index("s")
    wid = sid * 2 + cid
    w_base = wid * WPER

    pltpu.sync_copy(coef_hbm, coef_v)
    m00 = coef_v[0]
    m01 = coef_v[1]
    m02 = coef_v[2]
    m03 = coef_v[3]
    m10 = coef_v[4]
    m11 = coef_v[5]
    m12 = coef_v[6]
    m13 = coef_v[7]
    m20 = coef_v[8]
    m21 = coef_v[9]
    m22 = coef_v[10]
    m23 = coef_v[11]

    def copyin_start(k):
        b = k & 1
        base = w_base + k * CH
        return pltpu.async_copy(points_hbm.at[:, pl.ds(base, CH)],
                                uv_v[b], sem_in[b])

    def idx_stage(k):
        b = k & 1

        def body(i, c):
            o = i * L
            u = uv_v[b][0, pl.ds(o, L)]
            v = uv_v[b][1, pl.ds(o, L)]
            idx_v[b][pl.ds(o, L)] = (v << 11) + u
            return c

        lax.fori_loop(0, CH // L, body, 0)

    def gather_start(k):
        b = k & 1
        return pltpu.async_copy(depth_hbm.at[idx_v[b]], d_v[b], sem_g[b])

    def compute(k):
        b = k & 1

        def body(i, c):
            o = i * L
            pu = uv_v[b][0, pl.ds(o, L)].astype(jnp.float32)
            pv = uv_v[b][1, pl.ds(o, L)].astype(jnp.float32)
            d = d_v[b][pl.ds(o, L)]
            pd0 = pu * d
            pd1 = pv * d
            x = m00 * pd0 + m01 * pd1 + m02 * d + m03
            y = m10 * pd0 + m11 * pd1 + m12 * d + m13
            z = m20 * pd0 + m21 * pd1 + m22 * d + m23
            q_v[b][0, pl.ds(o, L)] = x / z
            q_v[b][1, pl.ds(o, L)] = y / z
            return c

        lax.fori_loop(0, CH // L, body, 0)

    def copyout_start(k):
        b = k & 1
        base = w_base + k * CH
        return pltpu.async_copy(q_v[b], out_hbm.at[:, pl.ds(base, CH)],
                                sem_out[b])

    ins = {}
    gs = {}
    outs = {}
    ins[0] = copyin_start(0)
    ins[0].wait()
    idx_stage(0)
    gs[0] = gather_start(0)
    ins[1] = copyin_start(1)
    for k in range(NCH):
        if k + 1 < NCH:
            ins[k + 1].wait()
            idx_stage(k + 1)
        gs[k].wait()
        if k + 1 < NCH:
            gs[k + 1] = gather_start(k + 1)
        if k - 2 >= 0:
            outs[k - 2].wait()
        compute(k)
        outs[k] = copyout_start(k)
        if k + 2 < NCH:
            ins[k + 2] = copyin_start(k + 2)
    for k in (NCH - 2, NCH - 1):
        outs[k].wait()

    @pl.when(wid == NW - 1)
    def _():
        n = REM_FULL
        pltpu.sync_copy(points_hbm.at[:, pl.ds(REM_BASE, n)],
                        uv_v[0].at[:, pl.ds(0, n)])

        def tbody(i, c):
            o = i * L
            u = uv_v[0][0, pl.ds(o, L)]
            v = uv_v[0][1, pl.ds(o, L)]
            idx_v[0][pl.ds(o, L)] = (v << 11) + u
            return c

        lax.fori_loop(0, n // L, tbody, 0)
        pltpu.async_copy(depth_hbm.at[idx_v[0].at[pl.ds(0, n)]],
                         d_v[0].at[pl.ds(0, n)], sem_g[0]).wait()

        def tbody2(i, c):
            o = i * L
            pu = uv_v[0][0, pl.ds(o, L)].astype(jnp.float32)
            pv = uv_v[0][1, pl.ds(o, L)].astype(jnp.float32)
            d = d_v[0][pl.ds(o, L)]
            pd0 = pu * d
            pd1 = pv * d
            x = m00 * pd0 + m01 * pd1 + m02 * d + m03
            y = m10 * pd0 + m11 * pd1 + m12 * d + m13
            z = m20 * pd0 + m21 * pd1 + m22 * d + m23
            q_v[0][0, pl.ds(o, L)] = x / z
            q_v[0][1, pl.ds(o, L)] = y / z
            return c

        lax.fori_loop(0, n // L, tbody2, 0)
        pltpu.sync_copy(q_v[0].at[:, pl.ds(0, n)],
                        out_hbm.at[:, pl.ds(REM_BASE, n)])

    @pl.when(wid == NW - 2)
    def _():
        pltpu.sync_copy(tail_uv_hbm, idx_v[0].at[pl.ds(0, 2 * REM_TAIL)])

        def t3(i, c):
            o = i * L
            u = idx_v[0][pl.ds(o, L)]
            v = idx_v[0][pl.ds(REM_TAIL + o, L)]
            idx_v[0][pl.ds(2 * REM_TAIL + o, L)] = (v << 11) + u
            return c

        lax.fori_loop(0, REM_TAIL // L, t3, 0)
        pltpu.async_copy(depth_hbm.at[idx_v[0].at[pl.ds(2 * REM_TAIL, REM_TAIL)]],
                         d_v[0].at[pl.ds(0, REM_TAIL)], sem_g[0]).wait()

        def t4(i, c):
            o = i * L
            pu = idx_v[0][pl.ds(o, L)].astype(jnp.float32)
            pv = idx_v[0][pl.ds(REM_TAIL + o, L)].astype(jnp.float32)
            d = d_v[0][pl.ds(o, L)]
            pd0 = pu * d
            pd1 = pv * d
            x = m00 * pd0 + m01 * pd1 + m02 * d + m03
            y = m10 * pd0 + m11 * pd1 + m12 * d + m13
            z = m20 * pd0 + m21 * pd1 + m22 * d + m23
            t_q[pl.ds(o, L)] = x / z
            t_q[pl.ds(REM_TAIL + o, L)] = y / z
            return c

        lax.fori_loop(0, REM_TAIL // L, t4, 0)
        pltpu.sync_copy(t_q, tailq_hbm)


@jax.jit
def _reproject(points, depth_flat, coef):
    f = functools.partial(
        pl.kernel,
        mesh=plsc.VectorSubcoreMesh(core_axis_name="c", subcore_axis_name="s"),
        out_type=(jax.ShapeDtypeStruct((2, N_PTS), jnp.float32),
                  jax.ShapeDtypeStruct((2 * REM_TAIL,), jnp.float32)),
        scratch_types=[
            pltpu.VMEM((2, CH), jnp.int32),
            pltpu.VMEM((2, CH), jnp.int32),
            pltpu.VMEM((CH,), jnp.int32),
            pltpu.VMEM((CH,), jnp.int32),
            pltpu.VMEM((CH,), jnp.float32),
            pltpu.VMEM((CH,), jnp.float32),
            pltpu.VMEM((2, CH), jnp.float32),
            pltpu.VMEM((2, CH), jnp.float32),
            pltpu.VMEM((13, 16), jnp.float32),
            pltpu.VMEM((2 * REM_TAIL,), jnp.float32),
            pltpu.SemaphoreType.DMA,
            pltpu.SemaphoreType.DMA,
            pltpu.SemaphoreType.DMA,
            pltpu.SemaphoreType.DMA,
            pltpu.SemaphoreType.DMA,
            pltpu.SemaphoreType.DMA,
        ],
    )(_tec_body)
    tail_uv = points[:, REM_BASE + REM_FULL:].reshape(2 * REM_TAIL)
    main_out, tail_q = f(points, depth_flat, coef, tail_uv)
    return lax.dynamic_update_slice(
        main_out, tail_q.reshape(2, REM_TAIL), (0, REM_BASE + REM_FULL))


def kernel(points, depth_img, odometry):
    M = jnp.linalg.inv(odometry)
    c13 = jnp.concatenate([M[:3, :].reshape(12).astype(jnp.float32),
                           jnp.ones((1,), jnp.float32)])
    coef = jnp.broadcast_to(c13.reshape(13, 1), (13, 16))
    depth_flat = depth_img.reshape(-1)
    return _reproject(points, depth_flat, coef)
